# baseline (speedup 1.0000x reference)
import jax
import jax.numpy as jnp
from jax import lax
from jax.experimental import pallas as pl
from jax.experimental.pallas import tpu as pltpu
from jax.experimental.pallas import tpu_sc as plsc

BATCH = 16384
WIDTH = 64
NW = 32
ROWS_PER_W = BATCH // NW
GCHUNK = 128
NCH = ROWS_PER_W // GCHUNK
NGROUPS = ROWS_PER_W // 16
EPS2 = 1e-16


def _body(hid_hbm, idx_hbm, proto_hbm, out_hbm,
          idx_v, hid_v, rows_v, out_v, sem_g, sem_h):
    cid = lax.axis_index("c")
    sid = lax.axis_index("s")
    wid = sid * 2 + cid
    base = wid * ROWS_PER_W

    pltpu.sync_copy(idx_hbm.at[wid], idx_v)
    hcopy = pltpu.async_copy(hid_hbm.at[pl.ds(base, ROWS_PER_W)], hid_v, sem_h)
    gcopies = [
        pltpu.async_copy(proto_hbm.at[idx_v.at[j]],
                         rows_v.at[pl.ds(j * GCHUNK, GCHUNK)], sem_g)
        for j in range(NCH)
    ]
    hcopy.wait()
    for c in gcopies:
        c.wait()

    iota = lax.iota(jnp.int32, 16)
    eps2 = jnp.full((16,), EPS2, jnp.float32)
    zero = jnp.zeros((16,), jnp.float32)
    magic = jnp.full((16,), 0x5F3759DF, jnp.int32)
    one = jnp.full((16,), 1, jnp.int32)
    c15 = jnp.full((16,), 1.5, jnp.float32)
    half = jnp.full((16,), 0.5, jnp.float32)

    def group(g, carry):
        row = iota + g * 16
        dot, hh, pp = zero, zero, zero
        for k in range(WIDTH):
            col = jnp.full((16,), k, jnp.int32)
            vh = plsc.load_gather(hid_v, [row, col])
            vp = plsc.load_gather(rows_v, [row, col])
            dot = dot + vh * vp
            hh = hh + vh * vh
            pp = pp + vp * vp
        d2 = jnp.maximum(hh, eps2) * jnp.maximum(pp, eps2)
        i = plsc.bitcast(d2, jnp.int32)
        i = magic - lax.shift_right_logical(i, one)
        y = plsc.bitcast(i, jnp.float32)
        for _ in range(3):
            y = y * (c15 - half * d2 * y * y)
        out_v[pl.ds(g * 16, 16)] = dot * y
        return carry

    lax.fori_loop(0, NGROUPS, group, 0)
    pltpu.sync_copy(out_v, out_hbm.at[pl.ds(base, ROWS_PER_W)])


def kernel(hidden, rel_ids, prototypes):
    idx = rel_ids.astype(jnp.int32).reshape(NW, NCH, GCHUNK)
    mesh = plsc.VectorSubcoreMesh(core_axis_name="c", subcore_axis_name="s")
    f = pl.kernel(
        _body,
        mesh=mesh,
        out_type=jax.ShapeDtypeStruct((BATCH,), jnp.float32),
        scratch_types=[
            pltpu.VMEM((NCH, GCHUNK), jnp.int32),
            pltpu.VMEM((ROWS_PER_W, WIDTH), jnp.float32),
            pltpu.VMEM((ROWS_PER_W, WIDTH), jnp.float32),
            pltpu.VMEM((ROWS_PER_W,), jnp.float32),
            pltpu.SemaphoreType.DMA,
            pltpu.SemaphoreType.DMA,
        ],
        compiler_params=pltpu.CompilerParams(
            needs_layout_passes=False, use_tc_tiling_on_sc=False),
    )
    return f(hidden, idx, prototypes)


# speedup vs baseline: 1.1062x; 1.1062x over previous
import jax
import jax.numpy as jnp
from jax import lax
from jax.experimental import pallas as pl
from jax.experimental.pallas import tpu as pltpu
from jax.experimental.pallas import tpu_sc as plsc

BATCH = 16384
WIDTH = 64
NW = 32
ROWS_PER_W = BATCH // NW
GCHUNK = 128
NCH = ROWS_PER_W // GCHUNK
GRP_PER_CH = GCHUNK // 16
EPS2 = 1e-16


def _body(hidT_hbm, pid_hbm, rel_hbm, proto2_hbm, out_hbm,
          pid_v, rel_v, hid_v, rows_v, out_v, sem_h, *sems):
    cid = lax.axis_index("c")
    sid = lax.axis_index("s")
    wid = sid * 2 + cid
    base = wid * ROWS_PER_W

    pltpu.sync_copy(pid_hbm.at[pl.ds(base, ROWS_PER_W)], pid_v)
    pltpu.sync_copy(rel_hbm.at[pl.ds(base, ROWS_PER_W)], rel_v)
    hcopy = pltpu.async_copy(hidT_hbm.at[:, pl.ds(base, ROWS_PER_W)],
                             hid_v, sem_h)
    gcopies = [
        pltpu.async_copy(proto2_hbm.at[pid_v.at[pl.ds(j * GCHUNK, GCHUNK)]],
                         rows_v.at[pl.ds(j * GCHUNK, GCHUNK)], sems[j])
        for j in range(NCH)
    ]
    hcopy.wait()

    iota = lax.iota(jnp.int32, 16)
    eps2 = jnp.full((16,), EPS2, jnp.float32)
    zero = jnp.zeros((16,), jnp.float32)
    one_i = jnp.full((16,), 1, jnp.int32)
    magic = jnp.full((16,), 0x5F3759DF, jnp.int32)
    c15 = jnp.full((16,), 1.5, jnp.float32)
    half = jnp.full((16,), 0.5, jnp.float32)

    def group(g, carry):
        row0 = g * 16
        rows16 = iota + row0
        rel16 = rel_v[pl.ds(row0, 16)]
        colb = (rel16 & one_i) * 64
        d0 = d1 = h0 = h1 = p0 = p1 = zero
        for c in range(WIDTH):
            cc = jnp.full((16,), c, jnp.int32)
            vh = hid_v[c, pl.ds(row0, 16)]
            vp = plsc.load_gather(rows_v, [rows16, colb + cc])
            if c % 2 == 0:
                d0 = d0 + vh * vp
                h0 = h0 + vh * vh
                p0 = p0 + vp * vp
            else:
                d1 = d1 + vh * vp
                h1 = h1 + vh * vh
                p1 = p1 + vp * vp
        dot, hh, pp = d0 + d1, h0 + h1, p0 + p1
        d2 = jnp.maximum(hh, eps2) * jnp.maximum(pp, eps2)
        i = magic - lax.shift_right_logical(plsc.bitcast(d2, jnp.int32), one_i)
        y = plsc.bitcast(i, jnp.float32)
        for _ in range(3):
            y = y * (c15 - half * d2 * y * y)
        out_v[pl.ds(row0, 16)] = dot * y
        return carry

    for j in range(NCH):
        gcopies[j].wait()
        lax.fori_loop(j * GRP_PER_CH, (j + 1) * GRP_PER_CH, group, 0)

    pltpu.sync_copy(out_v, out_hbm.at[pl.ds(base, ROWS_PER_W)])


def kernel(hidden, rel_ids, prototypes):
    rel = rel_ids.astype(jnp.int32)
    pid = lax.shift_right_logical(rel, 1)
    proto2 = prototypes.reshape(prototypes.shape[0] // 2, 2 * WIDTH)
    hidT = hidden.T
    mesh = plsc.VectorSubcoreMesh(core_axis_name="c", subcore_axis_name="s")
    f = pl.kernel(
        _body,
        mesh=mesh,
        out_type=jax.ShapeDtypeStruct((BATCH,), jnp.float32),
        scratch_types=[
            pltpu.VMEM((ROWS_PER_W,), jnp.int32),
            pltpu.VMEM((ROWS_PER_W,), jnp.int32),
            pltpu.VMEM((WIDTH, ROWS_PER_W), jnp.float32),
            pltpu.VMEM((ROWS_PER_W, 2 * WIDTH), jnp.float32),
            pltpu.VMEM((ROWS_PER_W,), jnp.float32),
            pltpu.SemaphoreType.DMA,
            pltpu.SemaphoreType.DMA,
            pltpu.SemaphoreType.DMA,
            pltpu.SemaphoreType.DMA,
            pltpu.SemaphoreType.DMA,
        ],
        compiler_params=pltpu.CompilerParams(
            needs_layout_passes=False, use_tc_tiling_on_sc=True),
    )
    return f(hidT, pid, rel, proto2)


# speedup vs baseline: 1.1741x; 1.0614x over previous
import jax
import jax.numpy as jnp
from jax import lax
from jax.experimental import pallas as pl
from jax.experimental.pallas import tpu as pltpu
from jax.experimental.pallas import tpu_sc as plsc

BATCH = 16384
WIDTH = 64
NW = 32
ROWS_PER_W = BATCH // NW
GCHUNK = 128
NCH = ROWS_PER_W // GCHUNK
GRP_PER_CH = GCHUNK // 16
EPS2 = 1e-16


def _body(hidT_hbm, pid_hbm, rel_hbm, proto2_hbm, out_hbm,
          pid_v, rel_v, hid_v, rows_v, out_v, sem_h, *sems):
    cid = lax.axis_index("c")
    sid = lax.axis_index("s")
    wid = sid * 2 + cid
    base = wid * ROWS_PER_W

    pltpu.sync_copy(pid_hbm.at[pl.ds(base, ROWS_PER_W)], pid_v)
    pltpu.sync_copy(rel_hbm.at[pl.ds(base, ROWS_PER_W)], rel_v)
    hcopy = pltpu.async_copy(hidT_hbm.at[:, pl.ds(base, ROWS_PER_W)],
                             hid_v, sem_h)
    gcopies = [
        pltpu.async_copy(proto2_hbm.at[pid_v.at[pl.ds(j * GCHUNK, GCHUNK)]],
                         rows_v.at[pl.ds(j * GCHUNK, GCHUNK)], sems[j])
        for j in range(NCH)
    ]
    hcopy.wait()

    iota = lax.iota(jnp.int32, 16)
    eps2 = jnp.full((16,), EPS2, jnp.float32)
    zero = jnp.zeros((16,), jnp.float32)
    one_i = jnp.full((16,), 1, jnp.int32)
    magic = jnp.full((16,), 0x5F3759DF, jnp.int32)
    c15 = jnp.full((16,), 1.5, jnp.float32)
    half = jnp.full((16,), 0.5, jnp.float32)

    def group(g):
        row0 = g * 16
        rows16 = iota + row0
        rel16 = rel_v[pl.ds(row0, 16)]
        col = (rel16 & one_i) * WIDTH
        d = [zero, zero, zero, zero]
        h = [zero, zero, zero, zero]
        p = [zero, zero, zero, zero]
        for c in range(WIDTH):
            vh = hid_v[c, pl.ds(row0, 16)]
            vp = plsc.load_gather(rows_v, [rows16, col])
            col = col + one_i
            a = c % 4
            d[a] = d[a] + vh * vp
            h[a] = h[a] + vh * vh
            p[a] = p[a] + vp * vp
        dot = (d[0] + d[1]) + (d[2] + d[3])
        hh = (h[0] + h[1]) + (h[2] + h[3])
        pp = (p[0] + p[1]) + (p[2] + p[3])
        d2 = jnp.maximum(hh, eps2) * jnp.maximum(pp, eps2)
        i = magic - lax.shift_right_logical(plsc.bitcast(d2, jnp.int32), one_i)
        y = plsc.bitcast(i, jnp.float32)
        for _ in range(3):
            y = y * (c15 - half * d2 * y * y)
        out_v[pl.ds(row0, 16)] = dot * y

    for j in range(NCH):
        gcopies[j].wait()
        plsc.parallel_loop(j * GRP_PER_CH, (j + 1) * GRP_PER_CH, 1,
                           unroll=2)(group)

    pltpu.sync_copy(out_v, out_hbm.at[pl.ds(base, ROWS_PER_W)])


def kernel(hidden, rel_ids, prototypes):
    rel = rel_ids.astype(jnp.int32)
    pid = lax.shift_right_logical(rel, 1)
    proto2 = prototypes.reshape(prototypes.shape[0] // 2, 2 * WIDTH)
    hidT = hidden.T
    mesh = plsc.VectorSubcoreMesh(core_axis_name="c", subcore_axis_name="s")
    f = pl.kernel(
        _body,
        mesh=mesh,
        out_type=jax.ShapeDtypeStruct((BATCH,), jnp.float32),
        scratch_types=[
            pltpu.VMEM((ROWS_PER_W,), jnp.int32),
            pltpu.VMEM((ROWS_PER_W,), jnp.int32),
            pltpu.VMEM((WIDTH, ROWS_PER_W), jnp.float32),
            pltpu.VMEM((ROWS_PER_W, 2 * WIDTH), jnp.float32),
            pltpu.VMEM((ROWS_PER_W,), jnp.float32),
            pltpu.SemaphoreType.DMA,
            pltpu.SemaphoreType.DMA,
            pltpu.SemaphoreType.DMA,
            pltpu.SemaphoreType.DMA,
            pltpu.SemaphoreType.DMA,
        ],
        compiler_params=pltpu.CompilerParams(
            needs_layout_passes=False, use_tc_tiling_on_sc=True),
    )
    return f(hidT, pid, rel, proto2)
